# baseline (speedup 1.0000x reference)
import functools

import jax
import jax.numpy as jnp
from jax import lax
from jax.experimental import pallas as pl
from jax.experimental.pallas import tpu as pltpu
from jax.experimental.pallas import tpu_sc as plsc

_ROWS = 1_000_000
_D = 32

_TC_ROWS = 520_768
_TB = 6_592
_TG = _TC_ROWS // _TB

_SC_ROWS = _ROWS - _TC_ROWS
_NW = 32
_CHUNK = 624
_NCHUNK = 24
_WROWS = _CHUNK * _NCHUNK
_GROUPS = _CHUNK // 16


def _scan_tc(q_ref, x_ref, outv_ref, outi_ref, best_ref, besti_ref):
    i = pl.program_id(0)

    @pl.when(i == 0)
    def _():
        best_ref[0] = jnp.inf
        besti_ref[0] = jnp.int32(0)

    x = x_ref[...]
    t = x - q_ref[...]
    s = jnp.sum(t * t, axis=1)
    m = jnp.min(s)
    ai = jnp.argmin(s).astype(jnp.int32)
    cond = m < best_ref[0]
    besti_ref[0] = jnp.where(cond, i * _TB + ai, besti_ref[0])
    best_ref[0] = jnp.where(cond, m, best_ref[0])

    @pl.when(i == _TG - 1)
    def _():
        outv_ref[0, 0] = best_ref[0]
        outi_ref[0, 0] = besti_ref[0]


def _scan_sc(q_hbm, x_hbm, outv_hbm, outi_hbm, qv, xbuf, stgv, stgi, sems, qsem):
    w = lax.axis_index("s") * 2 + lax.axis_index("c")
    row0 = w * _WROWS

    pltpu.async_copy(q_hbm.at[0], qv, qsem).wait()
    q1 = qv[pl.ds(0, 16)]
    q2 = qv[pl.ds(16, 16)]

    pltpu.make_async_copy(
        x_hbm.at[pl.ds(row0, _CHUNK)], xbuf.at[0], sems.at[0]).start()

    def row_sum(sel, r):
        v1 = xbuf[sel, r, pl.ds(0, 16)]
        v2 = xbuf[sel, r, pl.ds(16, 16)]
        t1 = v1 - q1
        t2 = v2 - q2
        return jnp.sum(t1 * t1 + t2 * t2)

    def chunk_body(c, carry):
        sel = lax.rem(c, 2)
        nxt = lax.rem(c + 1, 2)

        @pl.when(c + 1 < _NCHUNK)
        def _():
            pltpu.make_async_copy(
                x_hbm.at[pl.ds(row0 + (c + 1) * _CHUNK, _CHUNK)],
                xbuf.at[nxt], sems.at[nxt]).start()

        pltpu.make_async_copy(
            x_hbm.at[pl.ds(row0 + c * _CHUNK, _CHUNK)],
            xbuf.at[sel], sems.at[sel]).wait()

        base = _TC_ROWS + row0 + c * _CHUNK

        def group_body(g, carry2):
            best2, besti2 = carry2
            for j in range(16):
                r = g * 16 + j
                s = row_sum(sel, r)
                cond = s < best2
                besti2 = jnp.where(cond, base + r, besti2)
                best2 = jnp.where(cond, s, best2)
            return best2, besti2

        return lax.fori_loop(0, _GROUPS, group_body, carry)

    best, besti = lax.fori_loop(
        0, _NCHUNK, chunk_body, (jnp.float32(jnp.inf), jnp.int32(0)))

    stgv[...] = jnp.full((16,), best, jnp.float32)
    stgi[...] = jnp.full((16,), besti, jnp.int32)
    pltpu.async_copy(stgv, outv_hbm.at[w], sems.at[0]).wait()
    pltpu.async_copy(stgi, outi_hbm.at[w], sems.at[0]).wait()


def _merge_tc(scv_ref, sci_ref, tcv_ref, tci_ref, tt_ref, out_ref, sem):
    v = scv_ref[:, 0:1]
    mi = sci_ref[:, 0:1]
    tv = tcv_ref[0, 0]
    ti = tci_ref[0, 0]
    m = jnp.minimum(jnp.min(v), tv)
    big = jnp.int32(2**30)
    cand_sc = jnp.min(jnp.where(v == m, mi, big))
    cand_tc = jnp.where(tv == m, ti, big)
    best = jnp.minimum(cand_sc, cand_tc)
    cp = pltpu.make_async_copy(tt_ref.at[pl.ds(best, 1)], out_ref, sem)
    cp.start()
    cp.wait()


@jax.jit
def kernel(in_vel, train_obs_vel, train_target_vel):
    tcv, tci = pl.pallas_call(
        _scan_tc,
        grid=(_TG,),
        in_specs=[
            pl.BlockSpec((1, _D), lambda i: (0, 0)),
            pl.BlockSpec((_TB, _D), lambda i: (i, 0)),
        ],
        out_specs=[
            pl.BlockSpec(memory_space=pltpu.SMEM),
            pl.BlockSpec(memory_space=pltpu.SMEM),
        ],
        out_shape=[
            jax.ShapeDtypeStruct((1, 1), jnp.float32),
            jax.ShapeDtypeStruct((1, 1), jnp.int32),
        ],
        scratch_shapes=[
            pltpu.SMEM((1,), jnp.float32),
            pltpu.SMEM((1,), jnp.int32),
        ],
        compiler_params=pltpu.CompilerParams(
            dimension_semantics=("arbitrary",),
        ),
    )(in_vel, lax.slice(train_obs_vel, (0, 0), (_TC_ROWS, _D)))

    mesh = plsc.VectorSubcoreMesh(core_axis_name="c", subcore_axis_name="s")
    cp = pltpu.CompilerParams(
        needs_layout_passes=False, use_tc_tiling_on_sc=False)
    scan = functools.partial(
        pl.kernel,
        mesh=mesh,
        compiler_params=cp,
        out_type=[
            jax.ShapeDtypeStruct((_NW, 16), jnp.float32),
            jax.ShapeDtypeStruct((_NW, 16), jnp.int32),
        ],
        scratch_types=[
            pltpu.VMEM((32,), jnp.float32),
            pltpu.VMEM((2, _CHUNK, _D), jnp.float32),
            pltpu.VMEM((16,), jnp.float32),
            pltpu.VMEM((16,), jnp.int32),
            pltpu.SemaphoreType.DMA((2,)),
            pltpu.SemaphoreType.DMA,
        ],
    )(_scan_sc)
    scv, sci = scan(
        in_vel, lax.slice(train_obs_vel, (_TC_ROWS, 0), (_ROWS, _D)))

    out = pl.pallas_call(
        _merge_tc,
        in_specs=[
            pl.BlockSpec((_NW, 16), lambda: (0, 0)),
            pl.BlockSpec((_NW, 16), lambda: (0, 0)),
            pl.BlockSpec(memory_space=pltpu.SMEM),
            pl.BlockSpec(memory_space=pltpu.SMEM),
            pl.BlockSpec(memory_space=pl.MemorySpace.ANY),
        ],
        out_specs=pl.BlockSpec((1, _D), lambda: (0, 0)),
        out_shape=jax.ShapeDtypeStruct((1, _D), jnp.float32),
        scratch_shapes=[pltpu.SemaphoreType.DMA],
    )(scv, sci, tcv, tci, train_target_vel)
    return out[0]


# speedup vs baseline: 1.0850x; 1.0850x over previous
import dataclasses
import functools

import jax
import jax.numpy as jnp
from jax import lax
from jax.experimental import pallas as pl
from jax.experimental.pallas import tpu as pltpu
from jax.experimental.pallas import tpu_sc as plsc

_ROWS = 1_000_000
_D = 32
_NW = 32
_CHUNK = 1008
_NCHUNK = 31
_WROWS = _CHUNK * _NCHUNK
_TAIL = _ROWS - _NW * _WROWS
_GROUPS = _CHUNK // 16


def _scan_sc(q_hbm, x_hbm, outv_hbm, outi_hbm, qv, xbuf, stgv, stgi, sems, qsem):
    w = lax.axis_index("s") * 2 + lax.axis_index("c")
    row0 = w * _WROWS

    pltpu.async_copy(q_hbm.at[0], qv, qsem).wait()
    q1 = qv[pl.ds(0, 16)]
    q2 = qv[pl.ds(16, 16)]

    pltpu.make_async_copy(
        x_hbm.at[pl.ds(row0, _CHUNK)], xbuf.at[0], sems.at[0]).start()

    def row_sum(sel, r):
        v1 = xbuf[sel, r, pl.ds(0, 16)]
        v2 = xbuf[sel, r, pl.ds(16, 16)]
        t1 = v1 - q1
        t2 = v2 - q2
        return jnp.sum(t1 * t1 + t2 * t2)

    def chunk_body(c, carry):
        best, besti = carry
        sel = lax.rem(c, 2)
        nxt = lax.rem(c + 1, 2)

        @pl.when(c + 1 < _NCHUNK)
        def _():
            pltpu.make_async_copy(
                x_hbm.at[pl.ds(row0 + (c + 1) * _CHUNK, _CHUNK)],
                xbuf.at[nxt], sems.at[nxt]).start()

        pltpu.make_async_copy(
            x_hbm.at[pl.ds(row0 + c * _CHUNK, _CHUNK)],
            xbuf.at[sel], sems.at[sel]).wait()

        base = row0 + c * _CHUNK

        def group_body(g, carry2):
            best2, besti2 = carry2
            for j in range(16):
                r = g * 16 + j
                s = row_sum(sel, r)
                cond = s < best2
                besti2 = jnp.where(cond, base + r, besti2)
                best2 = jnp.where(cond, s, best2)
            return best2, besti2

        return lax.fori_loop(0, _GROUPS, group_body, (best, besti))

    best, besti = lax.fori_loop(
        0, _NCHUNK, chunk_body, (jnp.float32(jnp.inf), jnp.int32(0)))

    @pl.when(w == _NW - 1)
    def _():
        tbase = _NW * _WROWS
        tcp = pltpu.make_async_copy(
            x_hbm.at[pl.ds(tbase, _TAIL)], xbuf.at[0, pl.ds(0, _TAIL)],
            sems.at[0])
        tcp.start()
        tcp.wait()

    def tail_scan(carry):
        best3, besti3 = carry

        def tgroup(g, carry4):
            best4, besti4 = carry4
            for j in range(16):
                r = g * 16 + j
                s = row_sum(0, r)
                cond = s < best4
                besti4 = jnp.where(cond, _NW * _WROWS + r, besti4)
                best4 = jnp.where(cond, s, best4)
            return best4, besti4

        return lax.fori_loop(0, _TAIL // 16, tgroup, (best3, besti3))

    best, besti = lax.cond(w == _NW - 1, tail_scan, lambda c: c, (best, besti))

    stgv[...] = jnp.full((16,), best, jnp.float32)
    stgi[...] = jnp.full((16,), besti, jnp.int32)
    pltpu.async_copy(stgv, outv_hbm.at[w], sems.at[0]).wait()
    pltpu.async_copy(stgi, outi_hbm.at[w], sems.at[0]).wait()


def _merge_tc(outv_ref, outi_ref, tt_ref, out_ref, sem):
    v = outv_ref[:, 0:1]
    mi = outi_ref[:, 0:1]
    m = jnp.min(v)
    best = jnp.min(jnp.where(v == m, mi, jnp.int32(2**30)))
    cp = pltpu.make_async_copy(tt_ref.at[pl.ds(best, 1)], out_ref, sem)
    cp.start()
    cp.wait()


@jax.jit
def kernel(in_vel, train_obs_vel, train_target_vel):
    mesh = plsc.VectorSubcoreMesh(core_axis_name="c", subcore_axis_name="s")

    cp = pltpu.CompilerParams(
        needs_layout_passes=False, use_tc_tiling_on_sc=False)

    scan = functools.partial(
        pl.kernel,
        mesh=mesh,
        compiler_params=cp,
        out_type=[
            jax.ShapeDtypeStruct((_NW, 16), jnp.float32),
            jax.ShapeDtypeStruct((_NW, 16), jnp.int32),
        ],
        scratch_types=[
            pltpu.VMEM((32,), jnp.float32),
            pltpu.VMEM((2, _CHUNK, _D), jnp.float32),
            pltpu.VMEM((16,), jnp.float32),
            pltpu.VMEM((16,), jnp.int32),
            pltpu.SemaphoreType.DMA((2,)),
            pltpu.SemaphoreType.DMA,
        ],
    )(_scan_sc)

    outv, outi = scan(in_vel, train_obs_vel)

    out = pl.pallas_call(
        _merge_tc,
        in_specs=[
            pl.BlockSpec((_NW, 16), lambda: (0, 0)),
            pl.BlockSpec((_NW, 16), lambda: (0, 0)),
            pl.BlockSpec(memory_space=pl.MemorySpace.ANY),
        ],
        out_specs=pl.BlockSpec((1, _D), lambda: (0, 0)),
        out_shape=jax.ShapeDtypeStruct((1, _D), jnp.float32),
        scratch_shapes=[pltpu.SemaphoreType.DMA],
    )(outv, outi, train_target_vel)
    return out[0]
